# baseline (speedup 1.0000x reference)
import functools
import math

import jax
import jax.numpy as jnp
from jax import lax
from jax.experimental import pallas as pl
from jax.experimental.pallas import tpu as pltpu
from jax.experimental.pallas import tpu_sc as plsc

D = 64
ROW = 128
G = 4
NC, NS = 2, 16
NW = NC * NS
SCALE = math.sqrt(D)


def _sc_embed(x_rows, table):
    nrows = x_rows.shape[0]
    rows_per_w = nrows // NW
    ngroups = rows_per_w // G
    B = nrows * ROW

    mesh = plsc.VectorSubcoreMesh(core_axis_name="c", subcore_axis_name="s")

    @functools.partial(
        pl.kernel,
        mesh=mesh,
        out_type=jax.ShapeDtypeStruct((B, D), jnp.float32),
        scratch_types=[
            pltpu.VMEM((G, ROW), jnp.int32),
            pltpu.VMEM((G * ROW, D), jnp.float32),
            pltpu.SemaphoreType.DMA,
        ],
        compiler_params=pltpu.CompilerParams(use_tc_tiling_on_sc=False),
    )
    def k(x_hbm, table_hbm, out_hbm, idx_v, rows_v, sem):
        wid = lax.axis_index("s") * NC + lax.axis_index("c")
        row0 = wid * rows_per_w

        def group(g, carry):
            r0 = row0 + g * G
            pltpu.sync_copy(x_hbm.at[pl.ds(r0, G)], idx_v)
            cps = [
                pltpu.async_copy(
                    table_hbm.at[idx_v.at[j]],
                    rows_v.at[pl.ds(j * ROW, ROW)],
                    sem,
                )
                for j in range(G)
            ]
            for cp in cps:
                cp.wait()

            def scale_row(i, c):
                for kk in range(D // 16):
                    v = rows_v[i, pl.ds(kk * 16, 16)]
                    rows_v[i, pl.ds(kk * 16, 16)] = v * SCALE
                return c

            lax.fori_loop(0, G * ROW, scale_row, 0)
            pltpu.sync_copy(rows_v, out_hbm.at[pl.ds(r0 * ROW, G * ROW)])
            return carry

        lax.fori_loop(0, ngroups, group, 0)

    return k(x_rows, table)


def kernel(x, table):
    x_rows = x.reshape(-1, ROW).astype(jnp.int32)
    out = _sc_embed(x_rows, table)
    return out.reshape(x.shape[0], x.shape[1], D)


# speedup vs baseline: 1.1365x; 1.1365x over previous
import functools
import math

import jax
import jax.numpy as jnp
from jax import lax
from jax.experimental import pallas as pl
from jax.experimental.pallas import tpu as pltpu
from jax.experimental.pallas import tpu_sc as plsc

D = 64
ROW = 128
G = 2
NBUF = 4
NC, NS = 2, 16
NW = NC * NS
SCALE = math.sqrt(D)
GR = G * ROW


def _sc_embed(x_rows, table):
    nrows = x_rows.shape[0]
    rows_per_w = nrows // NW
    ngroups = rows_per_w // G
    B = nrows * ROW

    mesh = plsc.VectorSubcoreMesh(core_axis_name="c", subcore_axis_name="s")

    @functools.partial(
        pl.kernel,
        mesh=mesh,
        out_type=jax.ShapeDtypeStruct((B, D), jnp.float32),
        scratch_types=[
            pltpu.VMEM((rows_per_w, ROW), jnp.int32),
            [pltpu.VMEM((GR, D), jnp.float32) for _ in range(NBUF)],
            [pltpu.SemaphoreType.DMA for _ in range(NBUF)],
            [pltpu.SemaphoreType.DMA for _ in range(NBUF)],
        ],
        compiler_params=pltpu.CompilerParams(use_tc_tiling_on_sc=False),
    )
    def k(x_hbm, table_hbm, out_hbm, idx_v, rows, gsem, osem):
        wid = lax.axis_index("s") * NC + lax.axis_index("c")
        row0 = wid * rows_per_w
        out0 = row0 * ROW

        def gfire(g, s):
            for j in range(G):
                pltpu.async_copy(
                    table_hbm.at[idx_v.at[g * G + j]],
                    rows[s].at[pl.ds(j * ROW, ROW)],
                    gsem[s],
                )

        def gwait(s):
            pltpu.make_async_copy(
                out_hbm.at[pl.ds(0, GR)], rows[s], gsem[s]).wait()

        def ofire(g, s):
            pltpu.async_copy(
                rows[s], out_hbm.at[pl.ds(out0 + g * GR, GR)], osem[s])

        def owait(s):
            pltpu.make_async_copy(
                out_hbm.at[pl.ds(0, GR)], rows[s], osem[s]).wait()

        def scale(s):
            @pl.loop(0, GR, unroll=4)
            def _(i):
                for kk in range(D // 16):
                    v = rows[s][i, pl.ds(kk * 16, 16)]
                    rows[s][i, pl.ds(kk * 16, 16)] = v * SCALE

        def body(g, s, do_owait, do_prefetch):
            if do_owait:
                owait((s + 1) % NBUF)
            if do_prefetch:
                gfire(g + 1, (s + 1) % NBUF)
            gwait(s)
            scale(s)
            ofire(g, s)

        pltpu.sync_copy(x_hbm.at[pl.ds(row0, rows_per_w)], idx_v)
        gfire(0, 0)
        for g0 in range(NBUF):
            body(g0, g0 % NBUF, g0 >= NBUF - 1, True)

        @pl.loop(NBUF, ngroups - NBUF, step=NBUF)
        def _(p):
            for b in range(NBUF):
                body(p + b, b, True, True)

        for g0 in range(ngroups - NBUF, ngroups):
            body(g0, g0 % NBUF, True, g0 + 1 < ngroups)
        for g0 in range(ngroups - NBUF + 1, ngroups):
            owait(g0 % NBUF)

    return k(x_rows, table)


def kernel(x, table):
    x_rows = x.reshape(-1, ROW).astype(jnp.int32)
    out = _sc_embed(x_rows, table)
    return out.reshape(x.shape[0], x.shape[1], D)
